# Initial kernel scaffold; baseline (speedup 1.0000x reference)
import jax
import jax.numpy as jnp
from jax.experimental import pallas as pl


def kernel(x, edge_idx, W1_rel, b1, W1_root, W2_rel, b2, W2_root, W3_rel, b3, W3_root):
    raise NotImplementedError("write your pallas kernel here")


# speedup vs baseline: 3.1062x; 3.1062x over previous
import functools

import jax
import jax.numpy as jnp
from jax import lax
from jax.experimental import pallas as pl
from jax.experimental.pallas import tpu as pltpu
from jax.experimental.pallas import tpu_sc as plsc

_N = 10000
_E = 160000
_IND = 256
_HID = 512
_OUT = 128

_NC = 2
_NS = 16
_CE = 128
_EPAD = ((_E + _NC * _NS * _CE - 1) // (_NC * _NS * _CE)) * _NC * _NS * _CE
_N_ACC = 10112


def _make_sc_aggregate(nch, cw):
  mesh = plsc.VectorSubcoreMesh(core_axis_name="c", subcore_axis_name="s")
  npt = 624
  zr = _N_ACC // _NS
  edge_split = nch == 1
  ept = _EPAD // (_NC * _NS) if edge_split else _EPAD // _NS
  n_passes = 1 if edge_split else nch // _NC

  @functools.partial(
      pl.kernel,
      out_type=jax.ShapeDtypeStruct(((_NC if edge_split else nch) * _N, cw),
                                    jnp.float32),
      mesh=mesh,
      scratch_types=[
          pltpu.VMEM((ept,), jnp.int32),
          pltpu.VMEM((ept,), jnp.int32),
          pltpu.VMEM((_CE,), jnp.int32),
          pltpu.VMEM((_CE,), jnp.int32),
          pltpu.VMEM((_CE, cw), jnp.float32),
          pltpu.VMEM_SHARED((_N_ACC, cw), jnp.float32),
          pltpu.SemaphoreType.DMA,
      ],
  )
  def sc_agg(y_hbm, src_hbm, dst_hbm, zero_hbm, out_hbm,
             src_v, dst_v, gidx, didx, rows, acc, sem):
    cid = lax.axis_index("c")
    sid = lax.axis_index("s")
    ebase = (cid * _NS + sid) * ept if edge_split else sid * ept
    pltpu.sync_copy(src_hbm.at[pl.ds(ebase, ept)], src_v)
    pltpu.sync_copy(dst_hbm.at[pl.ds(ebase, ept)], dst_v)
    for p in range(n_passes):
      chunk = cid if edge_split else p * _NC + cid
      pltpu.sync_copy(zero_hbm.at[pl.ds(sid * zr, zr)],
                      acc.at[pl.ds(sid * zr, zr)])
      plsc.subcore_barrier()

      def body(k, carry):
        off = k * _CE
        for i in range(_CE // 16):
          s16 = src_v[pl.ds(off + i * 16, 16)]
          gidx[pl.ds(i * 16, 16)] = s16 if edge_split else s16 * nch + chunk
          didx[pl.ds(i * 16, 16)] = dst_v[pl.ds(off + i * 16, 16)]
        pltpu.async_copy(y_hbm.at[gidx], rows, sem).wait()
        pltpu.sync_copy(rows, acc.at[didx], add=True)
        return carry

      lax.fori_loop(0, ept // _CE, body, 0)
      plsc.subcore_barrier()
      pltpu.sync_copy(acc.at[pl.ds(sid * npt, npt)],
                      out_hbm.at[pl.ds(chunk * _N + sid * npt, npt)])

      @pl.when(sid == _NS - 1)
      def _drain_tail():
        pltpu.sync_copy(acc.at[pl.ds(_NS * npt, _N - _NS * npt)],
                        out_hbm.at[pl.ds(chunk * _N + _NS * npt, _N - _NS * npt)])

      plsc.subcore_barrier()

  return sc_agg


_sc_agg_l1 = _make_sc_aggregate(2, 128)
_sc_agg_l2 = _make_sc_aggregate(4, 128)
_sc_agg_l3 = _make_sc_aggregate(1, 128)

_BM = 1000


def _k1_body(agg_ref, x_ref, wrel_ref, wroot_ref, b_ref, out_ref):
  acc = jnp.dot(agg_ref[0], wrel_ref[0], preferred_element_type=jnp.float32)
  acc += jnp.dot(agg_ref[1], wrel_ref[1], preferred_element_type=jnp.float32)
  acc += jnp.dot(x_ref[...], wroot_ref[...], preferred_element_type=jnp.float32)
  acc += b_ref[...]
  out_ref[...] = jnp.maximum(acc, 0.0)


def _tc_layer1(agg1, x, w1rel, w1root, b1):
  return pl.pallas_call(
      _k1_body,
      grid=(_N // _BM,),
      in_specs=[
          pl.BlockSpec((2, _BM, 128), lambda m: (0, m, 0)),
          pl.BlockSpec((_BM, _IND), lambda m: (m, 0)),
          pl.BlockSpec((2, 128, _HID), lambda m: (0, 0, 0)),
          pl.BlockSpec((_IND, _HID), lambda m: (0, 0)),
          pl.BlockSpec((1, _HID), lambda m: (0, 0)),
      ],
      out_specs=pl.BlockSpec((_BM, _HID), lambda m: (m, 0)),
      out_shape=jax.ShapeDtypeStruct((_N, _HID), jnp.float32),
  )(agg1, x, w1rel, w1root, b1)


def _k2_body(agg_ref, h1_ref, w2rel_ref, w2root_ref, b2_ref,
             w3rel_ref, w3root_ref, y3_ref, r3_ref):
  acc = jnp.dot(h1_ref[...], w2root_ref[...], preferred_element_type=jnp.float32)
  for c in range(4):
    acc += jnp.dot(agg_ref[c], w2rel_ref[c], preferred_element_type=jnp.float32)
  h2 = jnp.maximum(acc + b2_ref[...], 0.0)
  y3_ref[...] = jnp.dot(h2, w3rel_ref[...], preferred_element_type=jnp.float32)
  r3_ref[...] = jnp.dot(h2, w3root_ref[...], preferred_element_type=jnp.float32)


def _tc_layer2(agg2, h1, w2rel, w2root, b2, w3rel, w3root):
  return pl.pallas_call(
      _k2_body,
      grid=(_N // _BM,),
      in_specs=[
          pl.BlockSpec((4, _BM, 128), lambda m: (0, m, 0)),
          pl.BlockSpec((_BM, _HID), lambda m: (m, 0)),
          pl.BlockSpec((4, 128, _HID), lambda m: (0, 0, 0)),
          pl.BlockSpec((_HID, _HID), lambda m: (0, 0)),
          pl.BlockSpec((1, _HID), lambda m: (0, 0)),
          pl.BlockSpec((_HID, _OUT), lambda m: (0, 0)),
          pl.BlockSpec((_HID, _OUT), lambda m: (0, 0)),
      ],
      out_specs=[
          pl.BlockSpec((_BM, _OUT), lambda m: (m, 0)),
          pl.BlockSpec((_BM, _OUT), lambda m: (m, 0)),
      ],
      out_shape=[
          jax.ShapeDtypeStruct((_N, _OUT), jnp.float32),
          jax.ShapeDtypeStruct((_N, _OUT), jnp.float32),
      ],
  )(agg2, h1, w2rel, w2root, b2, w3rel, w3root)


def _k3_body(agg_ref, r3_ref, b_ref, out_ref):
  z = agg_ref[0] + agg_ref[1] + r3_ref[...] + b_ref[...]
  z = z - jnp.max(z, axis=1, keepdims=True)
  e = jnp.exp(z)
  out_ref[...] = e / jnp.sum(e, axis=1, keepdims=True)


def _tc_layer3(agg3, r3, b3):
  return pl.pallas_call(
      _k3_body,
      grid=(_N // _BM,),
      in_specs=[
          pl.BlockSpec((2, _BM, _OUT), lambda m: (0, m, 0)),
          pl.BlockSpec((_BM, _OUT), lambda m: (m, 0)),
          pl.BlockSpec((1, _OUT), lambda m: (0, 0)),
      ],
      out_specs=pl.BlockSpec((_BM, _OUT), lambda m: (m, 0)),
      out_shape=jax.ShapeDtypeStruct((_N, _OUT), jnp.float32),
  )(agg3, r3, b3)


def kernel(x, edge_idx, W1_rel, b1, W1_root, W2_rel, b2, W2_root,
           W3_rel, b3, W3_root):
  src = edge_idx[0]
  dst = edge_idx[1]
  pad = _EPAD - _E
  src_p = jnp.concatenate([src, jnp.zeros((pad,), jnp.int32)])
  dst_p = jnp.concatenate([dst, jnp.full((pad,), _N, jnp.int32)])
  z128 = jnp.zeros((_N_ACC, 128), jnp.float32)

  agg1 = _sc_agg_l1(x.reshape(_N * 2, 128), src_p, dst_p, z128)
  h1 = _tc_layer1(agg1.reshape(2, _N, 128), x,
                  W1_rel.reshape(2, 128, _HID), W1_root, b1.reshape(1, _HID))
  agg2 = _sc_agg_l2(h1.reshape(_N * 4, 128), src_p, dst_p, z128)
  y3, r3 = _tc_layer2(agg2.reshape(4, _N, 128), h1,
                      W2_rel.reshape(4, 128, _HID), W2_root, b2.reshape(1, _HID),
                      W3_rel, W3_root)
  agg3 = _sc_agg_l3(y3, src_p, dst_p, z128)
  return _tc_layer3(agg3.reshape(2, _N, _OUT), r3, b3.reshape(1, _OUT))


# speedup vs baseline: 3.6578x; 1.1776x over previous
import functools

import jax
import jax.numpy as jnp
from jax import lax
from jax.experimental import pallas as pl
from jax.experimental.pallas import tpu as pltpu
from jax.experimental.pallas import tpu_sc as plsc

_N = 10000
_E = 160000
_IND = 256
_HID = 512
_OUT = 128

_NC = 2
_NS = 16
_CE = 128
_NBUF = 2
_EPAD = ((_E + _NC * _NS * _CE - 1) // (_NC * _NS * _CE)) * _NC * _NS * _CE
_N_ACC = 10112


def _make_sc_aggregate(nch, cw):
  mesh = plsc.VectorSubcoreMesh(core_axis_name="c", subcore_axis_name="s")
  npt = 624
  zr = _N_ACC // _NS
  edge_split = nch == 1
  ept = _EPAD // (_NC * _NS) if edge_split else _EPAD // _NS
  nct = ept // _CE
  n_passes = 1 if edge_split else nch // _NC

  @functools.partial(
      pl.kernel,
      out_type=jax.ShapeDtypeStruct(((_NC if edge_split else nch) * _N, cw),
                                    jnp.float32),
      mesh=mesh,
      scratch_types=[
          pltpu.VMEM((nct * _CE,), jnp.int32),
          [pltpu.VMEM((_CE,), jnp.int32) for _ in range(_NBUF)],
          [pltpu.VMEM((_CE,), jnp.int32) for _ in range(_NBUF)],
          [pltpu.VMEM((_CE, cw), jnp.float32) for _ in range(_NBUF)],
          pltpu.VMEM_SHARED((_N_ACC, cw), jnp.float32),
          [pltpu.SemaphoreType.DMA for _ in range(_NBUF)],
      ],
  )
  def sc_agg(y_hbm, ids_hbm, zero_hbm, out_hbm,
             ids_v, gidx_r, didx_r, rows_r, acc, sems):
    cid = lax.axis_index("c")
    sid = lax.axis_index("s")
    ebase = ((cid * _NS + sid) if edge_split else sid) * nct * _CE
    pltpu.sync_copy(ids_hbm.at[pl.ds(ebase, nct * _CE)], ids_v)
    ngrp = 4 if nct % 4 == 0 else 2
    gsz = nct // ngrp
    gmul = 1 if edge_split else nch

    for p in range(n_passes):
      chunk = cid if edge_split else p * _NC + cid
      gadd = 0 if edge_split else chunk
      pltpu.sync_copy(zero_hbm.at[pl.ds(sid * zr, zr)],
                      acc.at[pl.ds(sid * zr, zr)])
      plsc.subcore_barrier()

      def prep_fire(k, b):
        off = k * _CE
        for i in range(_CE // 16):
          m16 = ids_v[pl.ds(off + i * 16, 16)]
          didx_r[b][pl.ds(i * 16, 16)] = m16 >> 14
          gidx_r[b][pl.ds(i * 16, 16)] = (m16 & 16383) * gmul + gadd
        return pltpu.async_copy(y_hbm.at[gidx_r[b]], rows_r[b], sems[b])

      def group(g, carry):
        base = g * gsz
        descs = [None, None]
        descs[0] = prep_fire(base, 0)
        for j in range(gsz):
          b = j % 2
          if j + 1 < gsz:
            descs[b ^ 1] = prep_fire(base + j + 1, b ^ 1)
          descs[b].wait()
          pltpu.sync_copy(rows_r[b], acc.at[didx_r[b]], add=True)
        return carry

      lax.fori_loop(0, ngrp, group, 0)
      plsc.subcore_barrier()
      pltpu.sync_copy(acc.at[pl.ds(sid * npt, npt)],
                      out_hbm.at[pl.ds(chunk * _N + sid * npt, npt)])

      @pl.when(sid == _NS - 1)
      def _drain_tail():
        pltpu.sync_copy(acc.at[pl.ds(_NS * npt, _N - _NS * npt)],
                        out_hbm.at[pl.ds(chunk * _N + _NS * npt, _N - _NS * npt)])

      plsc.subcore_barrier()

  return sc_agg


_sc_agg_l1 = _make_sc_aggregate(2, 128)
_sc_agg_l2 = _make_sc_aggregate(4, 128)
_sc_agg_l3 = _make_sc_aggregate(1, 128)

_BM = 1000


def _k1_body(agg_ref, x_ref, wrel_ref, wroot_ref, b_ref, out_ref):
  acc = jnp.dot(agg_ref[0], wrel_ref[0], preferred_element_type=jnp.float32)
  acc += jnp.dot(agg_ref[1], wrel_ref[1], preferred_element_type=jnp.float32)
  acc += jnp.dot(x_ref[...], wroot_ref[...], preferred_element_type=jnp.float32)
  acc += b_ref[...]
  out_ref[...] = jnp.maximum(acc, 0.0)


def _tc_layer1(agg1, x, w1rel, w1root, b1):
  return pl.pallas_call(
      _k1_body,
      grid=(_N // _BM,),
      in_specs=[
          pl.BlockSpec((2, _BM, 128), lambda m: (0, m, 0)),
          pl.BlockSpec((_BM, _IND), lambda m: (m, 0)),
          pl.BlockSpec((2, 128, _HID), lambda m: (0, 0, 0)),
          pl.BlockSpec((_IND, _HID), lambda m: (0, 0)),
          pl.BlockSpec((1, _HID), lambda m: (0, 0)),
      ],
      out_specs=pl.BlockSpec((_BM, _HID), lambda m: (m, 0)),
      out_shape=jax.ShapeDtypeStruct((_N, _HID), jnp.float32),
  )(agg1, x, w1rel, w1root, b1)


def _k2_body(agg_ref, h1_ref, w2rel_ref, w2root_ref, b2_ref,
             w3rel_ref, w3root_ref, y3_ref, r3_ref):
  acc = jnp.dot(h1_ref[...], w2root_ref[...], preferred_element_type=jnp.float32)
  for c in range(4):
    acc += jnp.dot(agg_ref[c], w2rel_ref[c], preferred_element_type=jnp.float32)
  h2 = jnp.maximum(acc + b2_ref[...], 0.0)
  y3_ref[...] = jnp.dot(h2, w3rel_ref[...], preferred_element_type=jnp.float32)
  r3_ref[...] = jnp.dot(h2, w3root_ref[...], preferred_element_type=jnp.float32)


def _tc_layer2(agg2, h1, w2rel, w2root, b2, w3rel, w3root):
  return pl.pallas_call(
      _k2_body,
      grid=(_N // _BM,),
      in_specs=[
          pl.BlockSpec((4, _BM, 128), lambda m: (0, m, 0)),
          pl.BlockSpec((_BM, _HID), lambda m: (m, 0)),
          pl.BlockSpec((4, 128, _HID), lambda m: (0, 0, 0)),
          pl.BlockSpec((_HID, _HID), lambda m: (0, 0)),
          pl.BlockSpec((1, _HID), lambda m: (0, 0)),
          pl.BlockSpec((_HID, _OUT), lambda m: (0, 0)),
          pl.BlockSpec((_HID, _OUT), lambda m: (0, 0)),
      ],
      out_specs=[
          pl.BlockSpec((_BM, _OUT), lambda m: (m, 0)),
          pl.BlockSpec((_BM, _OUT), lambda m: (m, 0)),
      ],
      out_shape=[
          jax.ShapeDtypeStruct((_N, _OUT), jnp.float32),
          jax.ShapeDtypeStruct((_N, _OUT), jnp.float32),
      ],
  )(agg2, h1, w2rel, w2root, b2, w3rel, w3root)


def _k3_body(agg_ref, r3_ref, b_ref, out_ref):
  z = agg_ref[0] + agg_ref[1] + r3_ref[...] + b_ref[...]
  z = z - jnp.max(z, axis=1, keepdims=True)
  e = jnp.exp(z)
  out_ref[...] = e / jnp.sum(e, axis=1, keepdims=True)


def _tc_layer3(agg3, r3, b3):
  return pl.pallas_call(
      _k3_body,
      grid=(_N // _BM,),
      in_specs=[
          pl.BlockSpec((2, _BM, _OUT), lambda m: (0, m, 0)),
          pl.BlockSpec((_BM, _OUT), lambda m: (m, 0)),
          pl.BlockSpec((1, _OUT), lambda m: (0, 0)),
      ],
      out_specs=pl.BlockSpec((_BM, _OUT), lambda m: (m, 0)),
      out_shape=jax.ShapeDtypeStruct((_N, _OUT), jnp.float32),
  )(agg3, r3, b3)


def kernel(x, edge_idx, W1_rel, b1, W1_root, W2_rel, b2, W2_root,
           W3_rel, b3, W3_root):
  src = edge_idx[0]
  dst = edge_idx[1]
  pad = _EPAD - _E
  src_p = jnp.concatenate([src, jnp.zeros((pad,), jnp.int32)])
  dst_p = jnp.concatenate([dst, jnp.full((pad,), _N, jnp.int32)])
  ids_p = dst_p * 16384 + src_p
  z128 = jnp.zeros((_N_ACC, 128), jnp.float32)

  agg1 = _sc_agg_l1(x.reshape(_N * 2, 128), ids_p, z128)
  h1 = _tc_layer1(agg1.reshape(2, _N, 128), x,
                  W1_rel.reshape(2, 128, _HID), W1_root, b1.reshape(1, _HID))
  agg2 = _sc_agg_l2(h1.reshape(_N * 4, 128), ids_p, z128)
  y3, r3 = _tc_layer2(agg2.reshape(4, _N, 128), h1,
                      W2_rel.reshape(4, 128, _HID), W2_root, b2.reshape(1, _HID),
                      W3_rel, W3_root)
  agg3 = _sc_agg_l3(y3, ids_p, z128)
  return _tc_layer3(agg3.reshape(2, _N, _OUT), r3, b3.reshape(1, _OUT))
